# baseline (speedup 1.0000x reference)
import functools

import jax
import jax.numpy as jnp
from jax import lax
from jax.experimental import pallas as pl
from jax.experimental.pallas import tpu as pltpu
from jax.experimental.pallas import tpu_sc as plsc

VOCAB = 1000
D = 128
W = 10
P = 128
NC, NS = 2, 16
NW = NC * NS


def _scale_body(table_ref, out_ref):
    w = pl.program_id(0)
    scale = (10.0 - w.astype(jnp.float32)) / 55.0
    out_ref[...] = (table_ref[...] * scale)[None]


def _build_scaled(table):
    return pl.pallas_call(
        _scale_body,
        grid=(W,),
        in_specs=[pl.BlockSpec((VOCAB, D), lambda w: (0, 0))],
        out_specs=pl.BlockSpec((1, VOCAB, D), lambda w: (w, 0, 0)),
        out_shape=jax.ShapeDtypeStruct((W, VOCAB, D), jnp.float32),
    )(table)


CB = 100


def _idx_body(x_ref, idx_ref):
    offs = lax.broadcasted_iota(jnp.int32, (1, W, P), 1) * VOCAB
    idx_ref[...] = jnp.transpose(x_ref[...], (0, 2, 1)) + offs


def _build_idx(xf):
    n_chunks = xf.shape[0]
    return pl.pallas_call(
        _idx_body,
        grid=(n_chunks // CB,),
        in_specs=[pl.BlockSpec((CB, P, W), lambda i: (i, 0, 0))],
        out_specs=pl.BlockSpec((CB, W, P), lambda i: (i, 0, 0)),
        out_shape=jax.ShapeDtypeStruct((n_chunks, W, P), jnp.int32),
    )(xf)


NB = 4


def _make_sc_call(n_pos):
    n_chunks = n_pos // P
    cpw = n_chunks // NW
    mesh = plsc.VectorSubcoreMesh(
        core_axis_name="c", subcore_axis_name="s", num_cores=NC, num_subcores=NS
    )

    @functools.partial(
        pl.kernel,
        mesh=mesh,
        out_type=jax.ShapeDtypeStruct((n_pos, D), jnp.float32),
        scratch_types=(
            [pltpu.VMEM((W, P), jnp.int32) for _ in range(NB)]
            + [pltpu.VMEM((P, D), jnp.float32) for _ in range(NB)]
            + [pltpu.SemaphoreType.DMA for _ in range(3 * NB)]
        ),
    )
    def sc_kernel(idx_hbm, scaled_hbm, out_hbm, *scr):
        idxs = scr[0:NB]
        accs = scr[NB : 2 * NB]
        gsems = scr[2 * NB : 3 * NB]
        isems = scr[3 * NB : 4 * NB]
        osems = scr[4 * NB : 5 * NB]
        wid = lax.axis_index("s") * NC + lax.axis_index("c")
        base = wid * cpw

        def fire_idx(c, k):
            pltpu.async_copy(idx_hbm.at[c], idxs[k], isems[k])

        def wait_idx(k):
            pltpu.make_async_copy(idx_hbm.at[base], idxs[k], isems[k]).wait()

        def fire_ow(k):
            pltpu.async_copy(scaled_hbm.at[idxs[k].at[0]], accs[k], gsems[k])

        def fire_adds(k):
            for w in range(1, W):
                pltpu.async_copy(
                    scaled_hbm.at[idxs[k].at[w]], accs[k], gsems[k], add=True
                )

        def wait_g(k, n):
            for _ in range(n):
                pltpu.make_async_copy(
                    scaled_hbm.at[pl.ds(0, P)], accs[k], gsems[k]
                ).wait()

        def fire_out(c, k):
            pltpu.async_copy(accs[k], out_hbm.at[pl.ds(c * P, P)], osems[k])

        def wait_out(k):
            pltpu.make_async_copy(
                out_hbm.at[pl.ds(0, P)], accs[k], osems[k]
            ).wait()

        def body(c, k):
            km1 = (k - 1) % NB
            km2 = (k - 2) % NB
            wait_idx(k)
            wait_out(k)
            fire_ow(k)
            wait_g(km1, 1)
            fire_adds(km1)
            wait_g(km2, W - 1)
            fire_out(c - 2, km2)
            fire_idx(c + 2, km2)

        fire_idx(base + 0, 0)
        fire_idx(base + 1, 1)
        wait_idx(0)
        fire_ow(0)
        fire_idx(base + 2, 2)
        wait_idx(1)
        fire_ow(1)
        wait_g(0, 1)
        fire_adds(0)
        fire_idx(base + 3, 3)
        for k in (2, 3):
            wait_idx(k)
            fire_ow(k)
            wait_g(k - 1, 1)
            fire_adds(k - 1)
            wait_g(k - 2, W - 1)
            fire_out(base + k - 2, k - 2)
            fire_idx(base + k + 2, k - 2)

        def loop_body(j, carry):
            c0 = base + NB + NB * j
            for k in range(NB):
                body(c0 + k, k)
            return carry

        lax.fori_loop(0, (cpw - 2 * NB + 2) // NB, loop_body, 0)

        for k in (0, 1):
            c = base + cpw - 2 + k
            wait_idx(k)
            wait_out(k)
            fire_ow(k)
            wait_g((k - 1) % NB, 1)
            fire_adds((k - 1) % NB)
            wait_g((k - 2) % NB, W - 1)
            fire_out(c - 2, (k - 2) % NB)
        wait_g(1, 1)
        fire_adds(1)
        wait_g(0, W - 1)
        fire_out(base + cpw - 2, 0)
        wait_g(1, W - 1)
        fire_out(base + cpw - 1, 1)
        for k in (2, 3, 0, 1):
            wait_out(k)

    return sc_kernel


def kernel(x, table):
    B, T, _ = x.shape
    n_pos = B * T
    scaled = _build_scaled(table).reshape(W * VOCAB, D)
    xf = x.reshape(n_pos // P, P, W).astype(jnp.int32)
    idx = _build_idx(xf)
    out = _make_sc_call(n_pos)(idx, scaled)
    return out.reshape(B, T, D)


# speedup vs baseline: 1.4124x; 1.4124x over previous
import functools

import jax
import jax.numpy as jnp
from jax import lax
from jax.experimental import pallas as pl
from jax.experimental.pallas import tpu as pltpu
from jax.experimental.pallas import tpu_sc as plsc

VOCAB = 1000
D = 128
W = 10
T = 50
NC, NS = 2, 16
NW = NC * NS


def _scale_body(table_ref, out_ref):
    w = pl.program_id(0)
    scale = (10.0 - w.astype(jnp.float32)) / 55.0
    out_ref[...] = (table_ref[...] * scale)[None]


def _build_scaled(table):
    return pl.pallas_call(
        _scale_body,
        grid=(W,),
        in_specs=[pl.BlockSpec((VOCAB, D), lambda w: (0, 0))],
        out_specs=pl.BlockSpec((1, VOCAB, D), lambda w: (w, 0, 0)),
        out_shape=jax.ShapeDtypeStruct((W, VOCAB, D), jnp.float32),
    )(table)


NB = 4


def _make_sc_call(n_b):
    cpw = n_b // NW
    mesh = plsc.VectorSubcoreMesh(
        core_axis_name="c", subcore_axis_name="s", num_cores=NC, num_subcores=NS
    )

    @functools.partial(
        pl.kernel,
        mesh=mesh,
        out_type=jax.ShapeDtypeStruct((n_b, T, D), jnp.float32),
        scratch_types=(
            [pltpu.VMEM((W, T), jnp.int32) for _ in range(NB)]
            + [pltpu.VMEM((T, D), jnp.float32) for _ in range(NB)]
            + [pltpu.SemaphoreType.DMA for _ in range(3 * NB)]
        ),
    )
    def sc_kernel(idx_hbm, scaled_hbm, out_hbm, *scr):
        idxs = scr[0:NB]
        accs = scr[NB : 2 * NB]
        gsems = scr[2 * NB : 3 * NB]
        isems = scr[3 * NB : 4 * NB]
        osems = scr[4 * NB : 5 * NB]
        wid = lax.axis_index("s") * NC + lax.axis_index("c")
        base = wid * cpw

        def fire_idx(c, k):
            pltpu.async_copy(idx_hbm.at[c], idxs[k], isems[k])

        def wait_idx(k):
            pltpu.make_async_copy(idx_hbm.at[base], idxs[k], isems[k]).wait()

        def fire_ow(k):
            pltpu.async_copy(scaled_hbm.at[idxs[k].at[0]], accs[k], gsems[k])

        def fire_adds(k):
            for w in range(1, W):
                pltpu.async_copy(
                    scaled_hbm.at[idxs[k].at[w]], accs[k], gsems[k], add=True
                )

        def wait_g(k, n):
            for _ in range(n):
                pltpu.make_async_copy(
                    out_hbm.at[base], accs[k], gsems[k]
                ).wait()

        def fire_out(c, k):
            pltpu.async_copy(accs[k], out_hbm.at[c], osems[k])

        def wait_out(k):
            pltpu.make_async_copy(
                out_hbm.at[base], accs[k], osems[k]
            ).wait()

        def body(c, k):
            km1 = (k - 1) % NB
            km2 = (k - 2) % NB
            wait_idx(k)
            wait_out(k)
            fire_ow(k)
            wait_g(km1, 1)
            fire_adds(km1)
            wait_g(km2, W - 1)
            fire_out(c - 2, km2)
            fire_idx(c + 2, km2)

        fire_idx(base + 0, 0)
        fire_idx(base + 1, 1)
        wait_idx(0)
        fire_ow(0)
        fire_idx(base + 2, 2)
        wait_idx(1)
        fire_ow(1)
        wait_g(0, 1)
        fire_adds(0)
        fire_idx(base + 3, 3)
        for k in (2, 3):
            wait_idx(k)
            fire_ow(k)
            wait_g(k - 1, 1)
            fire_adds(k - 1)
            wait_g(k - 2, W - 1)
            fire_out(base + k - 2, k - 2)
            fire_idx(base + k + 2, k - 2)

        n_body = cpw - 6
        n_grp = n_body // NB

        def loop_body(j, carry):
            c0 = base + NB + NB * j
            for k in range(NB):
                body(c0 + k, k)
            return carry

        lax.fori_loop(0, n_grp, loop_body, 0)
        for i in range(n_body - n_grp * NB):
            crel = 4 + n_grp * NB + i
            body(base + crel, crel % NB)

        e0, e1 = (cpw - 2) % NB, (cpw - 1) % NB
        for crel, k in ((cpw - 2, e0), (cpw - 1, e1)):
            wait_idx(k)
            wait_out(k)
            fire_ow(k)
            wait_g((k - 1) % NB, 1)
            fire_adds((k - 1) % NB)
            wait_g((k - 2) % NB, W - 1)
            fire_out(base + crel - 2, (k - 2) % NB)
        wait_g(e1, 1)
        fire_adds(e1)
        wait_g(e0, W - 1)
        fire_out(base + cpw - 2, e0)
        wait_g(e1, W - 1)
        fire_out(base + cpw - 1, e1)
        for k in range(NB):
            wait_out(k)

    return sc_kernel


def kernel(x, table):
    B, Tdim, _ = x.shape
    scaled = _build_scaled(table).reshape(W * VOCAB, D)
    xi = x.astype(jnp.int32)
    offs = jnp.arange(W, dtype=jnp.int32) * VOCAB
    idx = jnp.transpose(xi, (0, 2, 1)) + offs[None, :, None]
    return _make_sc_call(B)(idx, scaled)
